# baseline (speedup 1.0000x reference)
import functools

import jax
import jax.numpy as jnp
from jax import lax
from jax.experimental import pallas as pl
from jax.experimental.pallas import tpu as pltpu
from jax.experimental.pallas import tpu_sc as plsc

K = 8192
D = 256
COMMIT = 0.25

BM = 256
BN = 512


W1 = 2736
W2 = 5472


def _blk_argmin(d, jcol):
    lmin = jnp.min(d, axis=1, keepdims=True)
    larg = jnp.min(jnp.where(d == lmin, jcol, K), axis=1, keepdims=True)
    return lmin, larg.astype(jnp.int32)


def _argmin_body(nn, flat_ref, embed_ref, fnorm_ref, enorm_ref,
                 idx_out_ref, loss_ref, wmin, widx):
    n = pl.program_id(1)
    mm = lax.dot_general(flat_ref[...], embed_ref[...],
                         (((1,), (1,)), ((), ())),
                         preferred_element_type=jnp.float32)
    dist = (fnorm_ref[...] + enorm_ref[...]) - 2.0 * mm
    base = n * BN
    jcol = base + lax.broadcasted_iota(jnp.int32, dist.shape, 1)

    @pl.when(n == 0)
    def _init():
        for w in range(3):
            wmin[w][...] = jnp.full((BM, 1), jnp.inf, jnp.float32)
            widx[w][...] = jnp.zeros((BM, 1), jnp.int32)

    def combine(w_id, lmin, larg):
        for w in range(3):
            upd = jnp.logical_and(w_id == w, lmin < wmin[w][...])
            wmin[w][...] = jnp.where(upd, lmin, wmin[w][...])
            widx[w][...] = jnp.where(upd, larg, widx[w][...])

    lo = base
    hi = base + BN - 1
    w_lo = jnp.where(lo >= W2, 2, jnp.where(lo >= W1, 1, 0))
    w_hi = jnp.where(hi >= W2, 2, jnp.where(hi >= W1, 1, 0))

    @pl.when(w_lo == w_hi)
    def _single():
        lmin, larg = _blk_argmin(dist, jcol)
        combine(w_lo, lmin, larg)

    @pl.when(w_lo != w_hi)
    def _split():
        bound = jnp.where(w_lo == 0, W1, W2)
        in_lo = jcol < bound
        lmin_a, larg_a = _blk_argmin(jnp.where(in_lo, dist, jnp.inf), jcol)
        lmin_b, larg_b = _blk_argmin(jnp.where(in_lo, jnp.inf, dist), jcol)
        combine(w_lo, lmin_a, larg_a)
        combine(w_lo + 1, lmin_b, larg_b)

    @pl.when(n == nn - 1)
    def _finalize():
        m0, i0 = wmin[0][...], widx[0][...]
        m1, i1 = wmin[1][...], widx[1][...]
        m2, i2 = wmin[2][...], widx[2][...]
        acc = m0.astype(jnp.bfloat16).astype(jnp.float32)
        idx, val = i0, m0
        upd1 = m1 < acc
        idx = jnp.where(upd1, i1, idx)
        val = jnp.where(upd1, m1, val)
        acc = jnp.where(upd1, m1.astype(jnp.bfloat16).astype(jnp.float32), acc)
        upd2 = m2 < acc
        idx = jnp.where(upd2, i2, idx)
        val = jnp.where(upd2, m2, val)
        idx_out_ref[...] = idx

        m = pl.program_id(0)
        blk_sum = jnp.sum(val)

        @pl.when(m == 0)
        def _first():
            loss_ref[0, 0] = blk_sum

        @pl.when(m > 0)
        def _rest():
            loss_ref[0, 0] += blk_sum


def _vq_argmin(flat, fnorm, embed_weight, enorm):
    m_total = flat.shape[0]
    nm, nn = m_total // BM, K // BN
    idx2d, loss2d = pl.pallas_call(
        functools.partial(_argmin_body, nn),
        grid=(nm, nn),
        in_specs=[
            pl.BlockSpec((BM, D), lambda m, n: (m, 0)),
            pl.BlockSpec((BN, D), lambda m, n: (n, 0)),
            pl.BlockSpec((BM, 1), lambda m, n: (m, 0)),
            pl.BlockSpec((1, BN), lambda m, n: (0, n)),
        ],
        out_specs=[
            pl.BlockSpec((BM, 1), lambda m, n: (m, 0)),
            pl.BlockSpec(memory_space=pltpu.SMEM),
        ],
        out_shape=[
            jax.ShapeDtypeStruct((m_total, 1), jnp.int32),
            jax.ShapeDtypeStruct((1, 1), jnp.float32),
        ],
        scratch_shapes=[
            [pltpu.VMEM((BM, 1), jnp.float32) for _ in range(3)],
            [pltpu.VMEM((BM, 1), jnp.int32) for _ in range(3)],
        ],
        compiler_params=pltpu.CompilerParams(
            dimension_semantics=("arbitrary", "arbitrary")),
    )(flat, embed_weight, fnorm, enorm)
    return idx2d, loss2d


def _make_sc_gather(b_total):
    info = plsc.get_sparse_core_info()
    ncores, nsub = info.num_cores, info.num_subcores
    nw = ncores * nsub
    b_per_w = b_total // nw
    chunk = 256
    nchunks = b_per_w // chunk
    mesh = plsc.VectorSubcoreMesh(core_axis_name="c", subcore_axis_name="s")

    @functools.partial(
        pl.kernel,
        mesh=mesh,
        out_type=jax.ShapeDtypeStruct((b_total, D), jnp.float32),
        scratch_types=[
            pltpu.VMEM((chunk,), jnp.int32),
            pltpu.VMEM((chunk, D), jnp.float32),
            pltpu.SemaphoreType.DMA,
        ],
    )
    def gather(table_hbm, idx_hbm, out_hbm, idx_v, rows_v, sem):
        wid = lax.axis_index("s") * ncores + lax.axis_index("c")
        base = wid * b_per_w
        for c in range(nchunks):
            off = base + c * chunk
            pltpu.sync_copy(idx_hbm.at[pl.ds(off, chunk)], idx_v)
            pltpu.async_copy(table_hbm.at[idx_v], rows_v, sem).wait()
            pltpu.sync_copy(rows_v, out_hbm.at[pl.ds(off, chunk)])

    return gather


def kernel(z_e, embed_weight):
    flat = z_e.reshape(-1, D)
    m_total = flat.shape[0]
    fnorm = jnp.sum(z_e ** 2, axis=2).reshape(-1, 1)
    enorm = jnp.sum(embed_weight ** 2, axis=1)[None, :]

    idx2d, loss2d = _vq_argmin(flat.astype(jnp.bfloat16), fnorm,
                               embed_weight.astype(jnp.bfloat16), enorm)
    indices = idx2d.reshape(m_total)

    quantized = _make_sc_gather(m_total)(embed_weight, indices)

    loss = loss2d.reshape(()) * ((1.0 + COMMIT) / (m_total * D))
    quantized_st = quantized.reshape(z_e.shape)
    return (quantized_st, loss, indices)
